# baseline (speedup 1.0000x reference)
import jax
import jax.numpy as jnp
from jax.experimental import pallas as pl
from jax.experimental.pallas import tpu as pltpu

D_IN = 256
D = 4
TOPK = 64
N_TOK = 1024


def _frame_body(x_ref, wk_ref, wq_ref, out_ref, s_ref):
    x = x_ref[0]
    k = jnp.dot(x, wk_ref[...], preferred_element_type=jnp.float32)
    q = jnp.dot(x, wq_ref[...], preferred_element_type=jnp.float32)
    h = jax.lax.dot_general(k, q, (((1,), (1,)), ((), ())),
                            preferred_element_type=jnp.float32)
    s_ref[...] = h * jnp.float32(1.0 / 16.0)
    s = s_ref[...]
    m = jnp.max(s, axis=1, keepdims=True)
    e = jnp.exp(s - m)
    z = jnp.sum(e, axis=1, keepdims=True)
    p = e / z
    a = jnp.sum(p, axis=0, keepdims=True)
    iota = jax.lax.broadcasted_iota(jnp.int32, (1, N_TOK), 1)

    def body(i, av):
        mx = jnp.max(av)
        idx = jnp.min(jnp.where(av == mx, iota, 2 * N_TOK))
        out_ref[0, pl.ds(i, 1), :] = x_ref[0, pl.ds(idx, 1), :]
        return jnp.where(iota == idx, -jnp.inf, av)

    jax.lax.fori_loop(0, TOPK, body, a)


def kernel(x, wk, wq):
    N, T, n, d_in = x.shape
    xf = x.reshape(N * T, n, d_in)
    out = pl.pallas_call(
        _frame_body,
        grid=(N * T,),
        in_specs=[
            pl.BlockSpec((1, n, d_in), lambda i: (i, 0, 0)),
            pl.BlockSpec((d_in, D), lambda i: (0, 0)),
            pl.BlockSpec((d_in, D), lambda i: (0, 0)),
        ],
        out_specs=pl.BlockSpec((1, TOPK, d_in), lambda i: (i, 0, 0)),
        out_shape=jax.ShapeDtypeStruct((N * T, TOPK, d_in), jnp.float32),
        scratch_shapes=[pltpu.VMEM((n, n), jnp.float32)],
    )(xf, wk, wq)
    return out.reshape(N, T, TOPK, d_in)


# speedup vs baseline: 7.7017x; 7.7017x over previous
import functools

import jax
import jax.numpy as jnp
from jax import lax
from jax.experimental import pallas as pl
from jax.experimental.pallas import tpu as pltpu
from jax.experimental.pallas import tpu_sc as plsc

D_IN = 256
D = 4
TOPK = 64
N_TOK = 1024
NFRAME = 32

NC = 2
NS = 16
L = 16
NCHUNK = N_TOK // L


def _a_body(x_ref, wk_ref, wq_ref, a_ref, s_ref):
    x = x_ref[0]
    k = jnp.dot(x, wk_ref[...], preferred_element_type=jnp.float32)
    q = jnp.dot(x, wq_ref[...], preferred_element_type=jnp.float32)
    h = jax.lax.dot_general(k, q, (((1,), (1,)), ((), ())),
                            preferred_element_type=jnp.float32)
    s_ref[...] = h * jnp.float32(1.0 / 16.0)
    s = s_ref[...]
    m = jnp.max(s, axis=1, keepdims=True)
    e = jnp.exp(s - m)
    z = jnp.sum(e, axis=1, keepdims=True)
    p = e / z
    a_ref[0] = jnp.sum(p, axis=0, keepdims=True)


def _sc_topk_gather(a_hbm, xf_hbm, out_hbm, a_v, cm_v, idx_v, rows_v, sem):
    f = lax.axis_index("s") * NC + lax.axis_index("c")
    iota16 = lax.iota(jnp.int32, L)
    lane0 = iota16 == 0
    neg_inf = jnp.full((L,), -jnp.inf, jnp.float32)

    pltpu.sync_copy(a_hbm.at[pl.ds(f * N_TOK, N_TOK)], a_v)

    for g in range(NCHUNK // L):
        base = (g * L + iota16) * L
        m_g = neg_inf
        for kk in range(L):
            m_g = jnp.maximum(m_g, plsc.load_gather(a_v, [base + kk]))
        cm_v[g * L:(g + 1) * L] = m_g

    def sel_body(i, carry):
        c0 = cm_v[0:16]
        c1 = cm_v[16:32]
        c2 = cm_v[32:48]
        c3 = cm_v[48:64]
        vm = jnp.maximum(jnp.maximum(c0, c1), jnp.maximum(c2, c3))
        gm = jnp.max(vm)
        gmv = jnp.full((L,), gm)
        f0 = plsc.all_reduce_ffs(c0 == gmv)
        f1 = plsc.all_reduce_ffs(c1 == gmv)
        f2 = plsc.all_reduce_ffs(c2 == gmv)
        f3 = plsc.all_reduce_ffs(c3 == gmv)
        c_star = jnp.where(
            f0 < L, f0,
            jnp.where(f1 < L, L + f1,
                      jnp.where(f2 < L, 2 * L + f2, 3 * L + f3)))
        cidx = c_star * L + iota16
        chunk = plsc.load_gather(a_v, [cidx])
        l_v = plsc.all_reduce_ffs(chunk == gmv)
        j_v = c_star * L + l_v
        plsc.store_scatter(idx_v, [jnp.full((L,), i, jnp.int32)],
                           f * N_TOK + j_v, mask=lane0)
        plsc.store_scatter(a_v, [j_v], neg_inf, mask=lane0)
        chunk2 = plsc.load_gather(a_v, [cidx])
        nm = jnp.max(chunk2)
        plsc.store_scatter(cm_v, [c_star], jnp.full((L,), nm), mask=lane0)
        return carry

    lax.fori_loop(0, TOPK, sel_body, jnp.int32(0))

    pltpu.async_copy(xf_hbm.at[idx_v], rows_v, sem).wait()
    pltpu.sync_copy(rows_v, out_hbm.at[pl.ds(f * TOPK, TOPK)])


def kernel(x, wk, wq):
    N, T, n, d_in = x.shape
    xf = x.reshape(N * T, n, d_in)
    a3 = pl.pallas_call(
        _a_body,
        grid=(N * T,),
        in_specs=[
            pl.BlockSpec((1, n, d_in), lambda i: (i, 0, 0)),
            pl.BlockSpec((d_in, D), lambda i: (0, 0)),
            pl.BlockSpec((d_in, D), lambda i: (0, 0)),
        ],
        out_specs=pl.BlockSpec((1, 1, n), lambda i: (i, 0, 0)),
        out_shape=jax.ShapeDtypeStruct((N * T, 1, n), jnp.float32),
        scratch_shapes=[pltpu.VMEM((n, n), jnp.float32)],
    )(xf, wk, wq)
    a_flat = a3.reshape(N * T * n)

    mesh = plsc.VectorSubcoreMesh(core_axis_name="c", subcore_axis_name="s")
    sc_fn = pl.kernel(
        _sc_topk_gather,
        out_type=jax.ShapeDtypeStruct((N * T * TOPK, d_in), jnp.float32),
        mesh=mesh,
        compiler_params=pltpu.CompilerParams(needs_layout_passes=False),
        scratch_types=[
            pltpu.VMEM((n,), jnp.float32),
            pltpu.VMEM((NCHUNK,), jnp.float32),
            pltpu.VMEM((TOPK,), jnp.int32),
            pltpu.VMEM((TOPK, d_in), jnp.float32),
            pltpu.SemaphoreType.DMA,
        ],
    )
    out = sc_fn(a_flat, xf.reshape(N * T * n, d_in))
    return out.reshape(N, T, TOPK, d_in)


# speedup vs baseline: 7.7104x; 1.0011x over previous
import functools

import jax
import jax.numpy as jnp
from jax import lax
from jax.experimental import pallas as pl
from jax.experimental.pallas import tpu as pltpu
from jax.experimental.pallas import tpu_sc as plsc

D_IN = 256
D = 4
TOPK = 64
N_TOK = 1024
NFRAME = 32

NC = 2
NS = 16
L = 16
NCHUNK = N_TOK // L


def _a_body(x_ref, wk_ref, wq_ref, a_ref, s_ref):
    x = x_ref[0]
    k = jnp.dot(x, wk_ref[...], preferred_element_type=jnp.float32)
    q = jnp.dot(x, wq_ref[...], preferred_element_type=jnp.float32)
    h = jax.lax.dot_general(k, q, (((1,), (1,)), ((), ())),
                            preferred_element_type=jnp.float32)
    s_ref[...] = h * jnp.float32(1.0 / 16.0)
    s = s_ref[...]
    m = jnp.max(s, axis=1, keepdims=True)
    e = jnp.exp(s - m)
    z = jnp.sum(e, axis=1, keepdims=True)
    p = e * (jnp.float32(1.0) / z)
    a_ref[0] = jnp.sum(p, axis=0, keepdims=True)


def _sc_topk_gather(a_hbm, xf_hbm, out_hbm, a_v, cm_v, idx_v, rows_v, sem):
    f = lax.axis_index("s") * NC + lax.axis_index("c")
    iota16 = lax.iota(jnp.int32, L)
    lane0 = iota16 == 0
    neg_inf = jnp.full((L,), -jnp.inf, jnp.float32)

    pltpu.sync_copy(a_hbm.at[pl.ds(f * N_TOK, N_TOK)], a_v)

    for g in range(NCHUNK // L):
        base = (g * L + iota16) * L
        m_g = neg_inf
        for kk in range(L):
            m_g = jnp.maximum(m_g, plsc.load_gather(a_v, [base + kk]))
        cm_v[g * L:(g + 1) * L] = m_g

    def sel_body(i, carry):
        c0 = cm_v[0:16]
        c1 = cm_v[16:32]
        c2 = cm_v[32:48]
        c3 = cm_v[48:64]
        vm = jnp.maximum(jnp.maximum(c0, c1), jnp.maximum(c2, c3))
        gm = jnp.max(vm)
        gmv = jnp.full((L,), gm)
        f0 = plsc.all_reduce_ffs(c0 == gmv)
        f1 = plsc.all_reduce_ffs(c1 == gmv)
        f2 = plsc.all_reduce_ffs(c2 == gmv)
        f3 = plsc.all_reduce_ffs(c3 == gmv)
        c_star = jnp.where(
            f0 < L, f0,
            jnp.where(f1 < L, L + f1,
                      jnp.where(f2 < L, 2 * L + f2, 3 * L + f3)))
        cidx = c_star * L + iota16
        chunk = plsc.load_gather(a_v, [cidx])
        l_v = plsc.all_reduce_ffs(chunk == gmv)
        j_v = c_star * L + l_v
        plsc.store_scatter(idx_v, [jnp.full((L,), i, jnp.int32)],
                           f * N_TOK + j_v, mask=lane0)
        plsc.store_scatter(a_v, [j_v], neg_inf, mask=lane0)
        chunk2 = plsc.load_gather(a_v, [cidx])
        nm = jnp.max(chunk2)
        plsc.store_scatter(cm_v, [c_star], jnp.full((L,), nm), mask=lane0)
        return carry

    lax.fori_loop(0, TOPK, sel_body, jnp.int32(0))

    pltpu.async_copy(xf_hbm.at[idx_v], rows_v, sem).wait()
    pltpu.sync_copy(rows_v, out_hbm.at[pl.ds(f * TOPK, TOPK)])


def kernel(x, wk, wq):
    N, T, n, d_in = x.shape
    xf = x.reshape(N * T, n, d_in)
    a3 = pl.pallas_call(
        _a_body,
        grid=(N * T,),
        in_specs=[
            pl.BlockSpec((1, n, d_in), lambda i: (i, 0, 0)),
            pl.BlockSpec((d_in, D), lambda i: (0, 0)),
            pl.BlockSpec((d_in, D), lambda i: (0, 0)),
        ],
        out_specs=pl.BlockSpec((1, 1, n), lambda i: (i, 0, 0)),
        out_shape=jax.ShapeDtypeStruct((N * T, 1, n), jnp.float32),
        scratch_shapes=[pltpu.VMEM((n, n), jnp.float32)],
    )(xf, wk, wq)
    a_flat = a3.reshape(N * T * n)

    mesh = plsc.VectorSubcoreMesh(core_axis_name="c", subcore_axis_name="s")
    sc_fn = pl.kernel(
        _sc_topk_gather,
        out_type=jax.ShapeDtypeStruct((N * T * TOPK, d_in), jnp.float32),
        mesh=mesh,
        compiler_params=pltpu.CompilerParams(needs_layout_passes=False),
        scratch_types=[
            pltpu.VMEM((n,), jnp.float32),
            pltpu.VMEM((NCHUNK,), jnp.float32),
            pltpu.VMEM((TOPK,), jnp.int32),
            pltpu.VMEM((TOPK, d_in), jnp.float32),
            pltpu.SemaphoreType.DMA,
        ],
    )
    out = sc_fn(a_flat, xf.reshape(N * T * n, d_in))
    return out.reshape(N, T, TOPK, d_in)


# speedup vs baseline: 8.0534x; 1.0445x over previous
import functools

import jax
import jax.numpy as jnp
from jax import lax
from jax.experimental import pallas as pl
from jax.experimental.pallas import tpu as pltpu
from jax.experimental.pallas import tpu_sc as plsc

D_IN = 256
D = 4
TOPK = 64
N_TOK = 1024
NFRAME = 32

NC = 2
NS = 16
L = 16
NCHUNK = N_TOK // L


def _a_body(x_ref, wk_ref, wq_ref, a_ref, s_ref):
    x = x_ref[0]
    k = jnp.dot(x, wk_ref[...], preferred_element_type=jnp.float32)
    q = jnp.dot(x, wq_ref[...], preferred_element_type=jnp.float32)
    ks = k * jnp.float32(1.0 / 16.0)
    s = jax.lax.dot_general(ks, q, (((1,), (1,)), ((), ())),
                            preferred_element_type=jnp.float32)
    s_ref[...] = s
    m = jnp.max(s, axis=1, keepdims=True)
    e = jnp.exp(s_ref[...] - m)
    z = jnp.sum(e, axis=1, keepdims=True)
    p = e * (jnp.float32(1.0) / z)
    a_ref[0] = jnp.sum(p, axis=0, keepdims=True)


def _sc_topk_gather(a_hbm, xf_hbm, out_hbm, a_v, cm_v, idx_v, rows_v, sem):
    f = lax.axis_index("s") * NC + lax.axis_index("c")
    iota16 = lax.iota(jnp.int32, L)
    lane0 = iota16 == 0
    neg_inf = jnp.full((L,), -jnp.inf, jnp.float32)

    pltpu.sync_copy(a_hbm.at[pl.ds(f * N_TOK, N_TOK)], a_v)

    for g in range(NCHUNK // L):
        base = (g * L + iota16) * L
        m_g = neg_inf
        for kk in range(L):
            m_g = jnp.maximum(m_g, plsc.load_gather(a_v, [base + kk]))
        cm_v[g * L:(g + 1) * L] = m_g

    def sel_body(i, carry):
        c0 = cm_v[0:16]
        c1 = cm_v[16:32]
        c2 = cm_v[32:48]
        c3 = cm_v[48:64]
        vm = jnp.maximum(jnp.maximum(c0, c1), jnp.maximum(c2, c3))
        gm = jnp.max(vm)
        gmv = jnp.full((L,), gm)
        f0 = plsc.all_reduce_ffs(c0 == gmv)
        f1 = plsc.all_reduce_ffs(c1 == gmv)
        f2 = plsc.all_reduce_ffs(c2 == gmv)
        f3 = plsc.all_reduce_ffs(c3 == gmv)
        c_star = jnp.where(
            f0 < L, f0,
            jnp.where(f1 < L, L + f1,
                      jnp.where(f2 < L, 2 * L + f2, 3 * L + f3)))
        cidx = c_star * L + iota16
        chunk = plsc.load_gather(a_v, [cidx])
        l_v = plsc.all_reduce_ffs(chunk == gmv)
        j_v = c_star * L + l_v
        plsc.store_scatter(idx_v, [jnp.full((L,), i, jnp.int32)],
                           f * N_TOK + j_v, mask=lane0)
        plsc.store_scatter(a_v, [j_v], neg_inf, mask=lane0)
        chunk2 = plsc.load_gather(a_v, [cidx])
        nm = jnp.max(chunk2)
        plsc.store_scatter(cm_v, [c_star], jnp.full((L,), nm), mask=lane0)
        return carry

    lax.fori_loop(0, TOPK, sel_body, jnp.int32(0))

    pltpu.async_copy(xf_hbm.at[idx_v], rows_v, sem).wait()
    pltpu.sync_copy(rows_v, out_hbm.at[pl.ds(f * TOPK, TOPK)])


def kernel(x, wk, wq):
    N, T, n, d_in = x.shape
    xf = x.reshape(N * T, n, d_in)
    a3 = pl.pallas_call(
        _a_body,
        grid=(N * T,),
        in_specs=[
            pl.BlockSpec((1, n, d_in), lambda i: (i, 0, 0)),
            pl.BlockSpec((d_in, D), lambda i: (0, 0)),
            pl.BlockSpec((d_in, D), lambda i: (0, 0)),
        ],
        out_specs=pl.BlockSpec((1, 1, n), lambda i: (i, 0, 0)),
        out_shape=jax.ShapeDtypeStruct((N * T, 1, n), jnp.float32),
        scratch_shapes=[pltpu.VMEM((n, n), jnp.float32)],
    )(xf, wk, wq)
    a_flat = a3.reshape(N * T * n)

    mesh = plsc.VectorSubcoreMesh(core_axis_name="c", subcore_axis_name="s")
    sc_fn = pl.kernel(
        _sc_topk_gather,
        out_type=jax.ShapeDtypeStruct((N * T * TOPK, d_in), jnp.float32),
        mesh=mesh,
        compiler_params=pltpu.CompilerParams(needs_layout_passes=False),
        scratch_types=[
            pltpu.VMEM((n,), jnp.float32),
            pltpu.VMEM((NCHUNK,), jnp.float32),
            pltpu.VMEM((TOPK,), jnp.int32),
            pltpu.VMEM((TOPK, d_in), jnp.float32),
            pltpu.SemaphoreType.DMA,
        ],
    )
    out = sc_fn(a_flat, xf.reshape(N * T * n, d_in))
    return out.reshape(N, T, TOPK, d_in)


# speedup vs baseline: 8.1343x; 1.0100x over previous
import functools

import jax
import jax.numpy as jnp
from jax import lax
from jax.experimental import pallas as pl
from jax.experimental.pallas import tpu as pltpu
from jax.experimental.pallas import tpu_sc as plsc

D_IN = 256
D = 4
TOPK = 64
N_TOK = 1024
NFRAME = 32

NC = 2
NS = 16
L = 16
NCHUNK = N_TOK // L


def _a_body(x_ref, wk_ref, wq_ref, a_ref, s_ref):
    x = x_ref[0]
    k = jnp.dot(x, wk_ref[...], preferred_element_type=jnp.float32)
    q = jnp.dot(x, wq_ref[...], preferred_element_type=jnp.float32)
    ks = k * jnp.float32(1.0 / 16.0)
    s = jax.lax.dot_general(ks, q, (((1,), (1,)), ((), ())),
                            preferred_element_type=jnp.float32)
    s_ref[...] = s
    m = jnp.max(s, axis=1, keepdims=True)
    e = jnp.exp(s_ref[...] - m)
    z = jnp.sum(e, axis=1, keepdims=True)
    p = e * (jnp.float32(1.0) / z)
    a_ref[0] = jnp.sum(p, axis=0, keepdims=True)


def _sc_topk_gather(a_hbm, xf_hbm, out_hbm, a_v, cm_v, idx_v, rows_v, sem):
    f = lax.axis_index("s") * NC + lax.axis_index("c")
    iota16 = lax.iota(jnp.int32, L)
    lane0 = iota16 == 0
    neg_inf = jnp.full((L,), -jnp.inf, jnp.float32)

    pltpu.sync_copy(a_hbm.at[pl.ds(f * N_TOK, N_TOK)], a_v)

    for g in range(NCHUNK // L):
        base = (g * L + iota16) * L
        m_g = neg_inf
        for kk in range(L):
            m_g = jnp.maximum(m_g, plsc.load_gather(a_v, [base + kk]))
        cm_v[g * L:(g + 1) * L] = m_g

    def sel_body(i, carry):
        c0 = cm_v[0:16]
        c1 = cm_v[16:32]
        c2 = cm_v[32:48]
        c3 = cm_v[48:64]
        vm = jnp.maximum(jnp.maximum(c0, c1), jnp.maximum(c2, c3))
        gm = jnp.max(vm)
        gmv = jnp.full((L,), gm)
        f0 = plsc.all_reduce_ffs(c0 == gmv)
        f1 = plsc.all_reduce_ffs(c1 == gmv)
        f2 = plsc.all_reduce_ffs(c2 == gmv)
        f3 = plsc.all_reduce_ffs(c3 == gmv)
        c_star = jnp.where(
            f0 < L, f0,
            jnp.where(f1 < L, L + f1,
                      jnp.where(f2 < L, 2 * L + f2, 3 * L + f3)))
        cidx = c_star * L + iota16
        chunk = plsc.load_gather(a_v, [cidx])
        l_v = plsc.all_reduce_ffs(chunk == gmv)
        j_v = c_star * L + l_v
        plsc.store_scatter(idx_v, [jnp.full((L,), i, jnp.int32)],
                           f * N_TOK + j_v, mask=lane0)
        plsc.store_scatter(a_v, [j_v], neg_inf, mask=lane0)
        chunk2 = plsc.load_gather(a_v, [cidx])
        nm = jnp.max(chunk2)
        plsc.store_scatter(cm_v, [c_star], jnp.full((L,), nm), mask=lane0)
        return carry

    lax.fori_loop(0, TOPK, sel_body, jnp.int32(0))

    pltpu.async_copy(xf_hbm.at[idx_v], rows_v, sem).wait()
    pltpu.sync_copy(rows_v, out_hbm.at[pl.ds(f * TOPK, TOPK)])


def kernel(x, wk, wq):
    N, T, n, d_in = x.shape
    xf = x.reshape(N * T, n, d_in)
    a3 = pl.pallas_call(
        _a_body,
        grid=(N * T,),
        in_specs=[
            pl.BlockSpec((1, n, d_in), lambda i: (i, 0, 0)),
            pl.BlockSpec((d_in, D), lambda i: (0, 0)),
            pl.BlockSpec((d_in, D), lambda i: (0, 0)),
        ],
        out_specs=pl.BlockSpec((1, 1, n), lambda i: (i, 0, 0)),
        out_shape=jax.ShapeDtypeStruct((N * T, 1, n), jnp.float32),
        scratch_shapes=[pltpu.VMEM((n, n), jnp.float32)],
        compiler_params=pltpu.CompilerParams(
            dimension_semantics=("parallel",)),
    )(xf, wk, wq)
    a_flat = a3.reshape(N * T * n)

    mesh = plsc.VectorSubcoreMesh(core_axis_name="c", subcore_axis_name="s")
    sc_fn = pl.kernel(
        _sc_topk_gather,
        out_type=jax.ShapeDtypeStruct((N * T * TOPK, d_in), jnp.float32),
        mesh=mesh,
        compiler_params=pltpu.CompilerParams(needs_layout_passes=False),
        scratch_types=[
            pltpu.VMEM((n,), jnp.float32),
            pltpu.VMEM((NCHUNK,), jnp.float32),
            pltpu.VMEM((TOPK,), jnp.int32),
            pltpu.VMEM((TOPK, d_in), jnp.float32),
            pltpu.SemaphoreType.DMA,
        ],
    )
    out = sc_fn(a_flat, xf.reshape(N * T * n, d_in))
    return out.reshape(N, T, TOPK, d_in)


# speedup vs baseline: 8.6311x; 1.0611x over previous
import functools

import jax
import jax.numpy as jnp
from jax import lax
from jax.experimental import pallas as pl
from jax.experimental.pallas import tpu as pltpu
from jax.experimental.pallas import tpu_sc as plsc

D_IN = 256
D = 4
TOPK = 64
N_TOK = 1024
NFRAME = 32

NC = 2
NS = 16
L = 16
NCHUNK = N_TOK // L


FPB = 2


def _a_body(x_ref, wk_ref, wq_ref, a_ref, s_ref):
    for b in range(FPB):
        x = x_ref[b]
        k = jnp.dot(x, wk_ref[...], preferred_element_type=jnp.float32)
        q = jnp.dot(x, wq_ref[...], preferred_element_type=jnp.float32)
        ks = k * jnp.float32(1.0 / 16.0)
        s = jax.lax.dot_general(ks, q, (((1,), (1,)), ((), ())),
                                preferred_element_type=jnp.float32)
        s_ref[...] = s
        m = jnp.max(s, axis=1, keepdims=True)
        e = jnp.exp(s_ref[...] - m)
        z = jnp.sum(e, axis=1, keepdims=True)
        p = e * (jnp.float32(1.0) / z)
        a_ref[b] = jnp.sum(p, axis=0, keepdims=True)


def _sc_topk_gather(a_hbm, xf_hbm, out_hbm, a_v, cm_v, idx_v, rows_v, sem):
    f = lax.axis_index("s") * NC + lax.axis_index("c")
    iota16 = lax.iota(jnp.int32, L)
    lane0 = iota16 == 0
    neg_inf = jnp.full((L,), -jnp.inf, jnp.float32)

    pltpu.sync_copy(a_hbm.at[pl.ds(f * N_TOK, N_TOK)], a_v)

    for g in range(NCHUNK // L):
        base = (g * L + iota16) * L
        m_g = neg_inf
        for kk in range(L):
            m_g = jnp.maximum(m_g, plsc.load_gather(a_v, [base + kk]))
        cm_v[g * L:(g + 1) * L] = m_g

    def sel_body(i, carry):
        c0 = cm_v[0:16]
        c1 = cm_v[16:32]
        c2 = cm_v[32:48]
        c3 = cm_v[48:64]
        vm = jnp.maximum(jnp.maximum(c0, c1), jnp.maximum(c2, c3))
        gm = jnp.max(vm)
        gmv = jnp.full((L,), gm)
        f0 = plsc.all_reduce_ffs(c0 == gmv)
        f1 = plsc.all_reduce_ffs(c1 == gmv)
        f2 = plsc.all_reduce_ffs(c2 == gmv)
        f3 = plsc.all_reduce_ffs(c3 == gmv)
        c_star = jnp.where(
            f0 < L, f0,
            jnp.where(f1 < L, L + f1,
                      jnp.where(f2 < L, 2 * L + f2, 3 * L + f3)))
        cidx = c_star * L + iota16
        chunk = plsc.load_gather(a_v, [cidx])
        l_v = plsc.all_reduce_ffs(chunk == gmv)
        j_v = c_star * L + l_v
        plsc.store_scatter(idx_v, [jnp.full((L,), i, jnp.int32)],
                           f * N_TOK + j_v, mask=lane0)
        plsc.store_scatter(a_v, [j_v], neg_inf, mask=lane0)
        chunk2 = plsc.load_gather(a_v, [cidx])
        nm = jnp.max(chunk2)
        plsc.store_scatter(cm_v, [c_star], jnp.full((L,), nm), mask=lane0)
        return carry

    lax.fori_loop(0, TOPK, sel_body, jnp.int32(0))

    pltpu.async_copy(xf_hbm.at[idx_v], rows_v, sem).wait()
    pltpu.sync_copy(rows_v, out_hbm.at[pl.ds(f * TOPK, TOPK)])


def kernel(x, wk, wq):
    N, T, n, d_in = x.shape
    xf = x.reshape(N * T, n, d_in)
    a3 = pl.pallas_call(
        _a_body,
        grid=(N * T // FPB,),
        in_specs=[
            pl.BlockSpec((FPB, n, d_in), lambda i: (i, 0, 0)),
            pl.BlockSpec((d_in, D), lambda i: (0, 0)),
            pl.BlockSpec((d_in, D), lambda i: (0, 0)),
        ],
        out_specs=pl.BlockSpec((FPB, 1, n), lambda i: (i, 0, 0)),
        out_shape=jax.ShapeDtypeStruct((N * T, 1, n), jnp.float32),
        scratch_shapes=[pltpu.VMEM((n, n), jnp.float32)],
        compiler_params=pltpu.CompilerParams(
            dimension_semantics=("parallel",)),
    )(xf, wk, wq)
    a_flat = a3.reshape(N * T * n)

    mesh = plsc.VectorSubcoreMesh(core_axis_name="c", subcore_axis_name="s")
    sc_fn = pl.kernel(
        _sc_topk_gather,
        out_type=jax.ShapeDtypeStruct((N * T * TOPK, d_in), jnp.float32),
        mesh=mesh,
        compiler_params=pltpu.CompilerParams(needs_layout_passes=False),
        scratch_types=[
            pltpu.VMEM((n,), jnp.float32),
            pltpu.VMEM((NCHUNK,), jnp.float32),
            pltpu.VMEM((TOPK,), jnp.int32),
            pltpu.VMEM((TOPK, d_in), jnp.float32),
            pltpu.SemaphoreType.DMA,
        ],
    )
    out = sc_fn(a_flat, xf.reshape(N * T * n, d_in))
    return out.reshape(N, T, TOPK, d_in)


# speedup vs baseline: 9.0347x; 1.0468x over previous
import functools

import jax
import jax.numpy as jnp
from jax import lax
from jax.experimental import pallas as pl
from jax.experimental.pallas import tpu as pltpu
from jax.experimental.pallas import tpu_sc as plsc

D_IN = 256
D = 4
TOPK = 64
N_TOK = 1024
NFRAME = 32

NC = 2
NS = 16
L = 16
NCHUNK = N_TOK // L


FPB = 4


def _a_body(x_ref, wk_ref, wq_ref, a_ref, s_ref):
    for b in range(FPB):
        x = x_ref[b]
        k = jnp.dot(x, wk_ref[...], preferred_element_type=jnp.float32)
        q = jnp.dot(x, wq_ref[...], preferred_element_type=jnp.float32)
        ks = k * jnp.float32(1.0 / 16.0)
        s = jax.lax.dot_general(ks, q, (((1,), (1,)), ((), ())),
                                preferred_element_type=jnp.float32)
        s_ref[...] = s
        m = jnp.max(s, axis=1, keepdims=True)
        e = jnp.exp(s_ref[...] - m)
        z = jnp.sum(e, axis=1, keepdims=True)
        p = e * (jnp.float32(1.0) / z)
        a_ref[b] = jnp.sum(p, axis=0, keepdims=True)


def _sc_topk_gather(a_hbm, xf_hbm, out_hbm, a_v, cm_v, idx_v, rows_v, sem):
    f = lax.axis_index("s") * NC + lax.axis_index("c")
    iota16 = lax.iota(jnp.int32, L)
    lane0 = iota16 == 0
    neg_inf = jnp.full((L,), -jnp.inf, jnp.float32)

    pltpu.sync_copy(a_hbm.at[pl.ds(f * N_TOK, N_TOK)], a_v)

    for g in range(NCHUNK // L):
        base = (g * L + iota16) * L
        m_g = neg_inf
        for kk in range(L):
            m_g = jnp.maximum(m_g, plsc.load_gather(a_v, [base + kk]))
        cm_v[g * L:(g + 1) * L] = m_g

    def sel_body(i, carry):
        c0 = cm_v[0:16]
        c1 = cm_v[16:32]
        c2 = cm_v[32:48]
        c3 = cm_v[48:64]
        vm = jnp.maximum(jnp.maximum(c0, c1), jnp.maximum(c2, c3))
        gm = jnp.max(vm)
        gmv = jnp.full((L,), gm)
        f0 = plsc.all_reduce_ffs(c0 == gmv)
        f1 = plsc.all_reduce_ffs(c1 == gmv)
        f2 = plsc.all_reduce_ffs(c2 == gmv)
        f3 = plsc.all_reduce_ffs(c3 == gmv)
        c_star = jnp.where(
            f0 < L, f0,
            jnp.where(f1 < L, L + f1,
                      jnp.where(f2 < L, 2 * L + f2, 3 * L + f3)))
        cidx = c_star * L + iota16
        chunk = plsc.load_gather(a_v, [cidx])
        l_v = plsc.all_reduce_ffs(chunk == gmv)
        j_v = c_star * L + l_v
        plsc.store_scatter(idx_v, [jnp.full((L,), i, jnp.int32)],
                           f * N_TOK + j_v, mask=lane0)
        plsc.store_scatter(a_v, [j_v], neg_inf, mask=lane0)
        chunk2 = plsc.load_gather(a_v, [cidx])
        nm = jnp.max(chunk2)
        plsc.store_scatter(cm_v, [c_star], jnp.full((L,), nm), mask=lane0)
        return carry

    lax.fori_loop(0, TOPK, sel_body, jnp.int32(0))

    pltpu.async_copy(xf_hbm.at[idx_v], rows_v, sem).wait()
    pltpu.sync_copy(rows_v, out_hbm.at[pl.ds(f * TOPK, TOPK)])


def kernel(x, wk, wq):
    N, T, n, d_in = x.shape
    xf = x.reshape(N * T, n, d_in)
    a3 = pl.pallas_call(
        _a_body,
        grid=(N * T // FPB,),
        in_specs=[
            pl.BlockSpec((FPB, n, d_in), lambda i: (i, 0, 0)),
            pl.BlockSpec((d_in, D), lambda i: (0, 0)),
            pl.BlockSpec((d_in, D), lambda i: (0, 0)),
        ],
        out_specs=pl.BlockSpec((FPB, 1, n), lambda i: (i, 0, 0)),
        out_shape=jax.ShapeDtypeStruct((N * T, 1, n), jnp.float32),
        scratch_shapes=[pltpu.VMEM((n, n), jnp.float32)],
        compiler_params=pltpu.CompilerParams(
            dimension_semantics=("parallel",)),
    )(xf, wk, wq)
    a_flat = a3.reshape(N * T * n)

    mesh = plsc.VectorSubcoreMesh(core_axis_name="c", subcore_axis_name="s")
    sc_fn = pl.kernel(
        _sc_topk_gather,
        out_type=jax.ShapeDtypeStruct((N * T * TOPK, d_in), jnp.float32),
        mesh=mesh,
        compiler_params=pltpu.CompilerParams(needs_layout_passes=False),
        scratch_types=[
            pltpu.VMEM((n,), jnp.float32),
            pltpu.VMEM((NCHUNK,), jnp.float32),
            pltpu.VMEM((TOPK,), jnp.int32),
            pltpu.VMEM((TOPK, d_in), jnp.float32),
            pltpu.SemaphoreType.DMA,
        ],
    )
    out = sc_fn(a_flat, xf.reshape(N * T * n, d_in))
    return out.reshape(N, T, TOPK, d_in)


# speedup vs baseline: 9.1372x; 1.0113x over previous
import functools

import jax
import jax.numpy as jnp
from jax import lax
from jax.experimental import pallas as pl
from jax.experimental.pallas import tpu as pltpu
from jax.experimental.pallas import tpu_sc as plsc

D_IN = 256
D = 4
TOPK = 64
N_TOK = 1024
NFRAME = 32

NC = 2
NS = 16
L = 16
NCHUNK = N_TOK // L


FPB = 8


def _a_body(x_ref, wk_ref, wq_ref, a_ref, s_ref):
    for b in range(FPB):
        x = x_ref[b]
        k = jnp.dot(x, wk_ref[...], preferred_element_type=jnp.float32)
        q = jnp.dot(x, wq_ref[...], preferred_element_type=jnp.float32)
        ks = k * jnp.float32(1.0 / 16.0)
        s = jax.lax.dot_general(ks, q, (((1,), (1,)), ((), ())),
                                preferred_element_type=jnp.float32)
        s_ref[...] = s
        m = jnp.max(s, axis=1, keepdims=True)
        e = jnp.exp(s_ref[...] - m)
        z = jnp.sum(e, axis=1, keepdims=True)
        p = e * (jnp.float32(1.0) / z)
        a_ref[b] = jnp.sum(p, axis=0, keepdims=True)


def _sc_topk_gather(a_hbm, xf_hbm, out_hbm, a_v, cm_v, idx_v, rows_v, sem):
    f = lax.axis_index("s") * NC + lax.axis_index("c")
    iota16 = lax.iota(jnp.int32, L)
    lane0 = iota16 == 0
    neg_inf = jnp.full((L,), -jnp.inf, jnp.float32)

    pltpu.sync_copy(a_hbm.at[pl.ds(f * N_TOK, N_TOK)], a_v)

    for g in range(NCHUNK // L):
        base = (g * L + iota16) * L
        m_g = neg_inf
        for kk in range(L):
            m_g = jnp.maximum(m_g, plsc.load_gather(a_v, [base + kk]))
        cm_v[g * L:(g + 1) * L] = m_g

    def sel_body(i, carry):
        c0 = cm_v[0:16]
        c1 = cm_v[16:32]
        c2 = cm_v[32:48]
        c3 = cm_v[48:64]
        vm = jnp.maximum(jnp.maximum(c0, c1), jnp.maximum(c2, c3))
        gm = jnp.max(vm)
        gmv = jnp.full((L,), gm)
        f0 = plsc.all_reduce_ffs(c0 == gmv)
        f1 = plsc.all_reduce_ffs(c1 == gmv)
        f2 = plsc.all_reduce_ffs(c2 == gmv)
        f3 = plsc.all_reduce_ffs(c3 == gmv)
        c_star = jnp.where(
            f0 < L, f0,
            jnp.where(f1 < L, L + f1,
                      jnp.where(f2 < L, 2 * L + f2, 3 * L + f3)))
        cidx = c_star * L + iota16
        chunk = plsc.load_gather(a_v, [cidx])
        l_v = plsc.all_reduce_ffs(chunk == gmv)
        j_v = c_star * L + l_v
        plsc.store_scatter(idx_v, [jnp.full((L,), i, jnp.int32)],
                           f * N_TOK + j_v, mask=lane0)
        plsc.store_scatter(a_v, [j_v], neg_inf, mask=lane0)
        chunk2 = plsc.load_gather(a_v, [cidx])
        nm = jnp.max(chunk2)
        plsc.store_scatter(cm_v, [c_star], jnp.full((L,), nm), mask=lane0)
        return carry

    lax.fori_loop(0, TOPK, sel_body, jnp.int32(0))

    pltpu.async_copy(xf_hbm.at[idx_v], rows_v, sem).wait()
    pltpu.sync_copy(rows_v, out_hbm.at[pl.ds(f * TOPK, TOPK)])


def kernel(x, wk, wq):
    N, T, n, d_in = x.shape
    xf = x.reshape(N * T, n, d_in)
    a3 = pl.pallas_call(
        _a_body,
        grid=(N * T // FPB,),
        in_specs=[
            pl.BlockSpec((FPB, n, d_in), lambda i: (i, 0, 0)),
            pl.BlockSpec((d_in, D), lambda i: (0, 0)),
            pl.BlockSpec((d_in, D), lambda i: (0, 0)),
        ],
        out_specs=pl.BlockSpec((FPB, 1, n), lambda i: (i, 0, 0)),
        out_shape=jax.ShapeDtypeStruct((N * T, 1, n), jnp.float32),
        scratch_shapes=[pltpu.VMEM((n, n), jnp.float32)],
        compiler_params=pltpu.CompilerParams(
            dimension_semantics=("parallel",)),
    )(xf, wk, wq)
    a_flat = a3.reshape(N * T * n)

    mesh = plsc.VectorSubcoreMesh(core_axis_name="c", subcore_axis_name="s")
    sc_fn = pl.kernel(
        _sc_topk_gather,
        out_type=jax.ShapeDtypeStruct((N * T * TOPK, d_in), jnp.float32),
        mesh=mesh,
        compiler_params=pltpu.CompilerParams(needs_layout_passes=False),
        scratch_types=[
            pltpu.VMEM((n,), jnp.float32),
            pltpu.VMEM((NCHUNK,), jnp.float32),
            pltpu.VMEM((TOPK,), jnp.int32),
            pltpu.VMEM((TOPK, d_in), jnp.float32),
            pltpu.SemaphoreType.DMA,
        ],
    )
    out = sc_fn(a_flat, xf.reshape(N * T * n, d_in))
    return out.reshape(N, T, TOPK, d_in)
